# Initial kernel scaffold; baseline (speedup 1.0000x reference)
import jax
import jax.numpy as jnp
from jax.experimental import pallas as pl


def kernel(loc_data, conf_data, anchors, targets):
    raise NotImplementedError("write your pallas kernel here")


# speedup vs baseline: 32.8671x; 32.8671x over previous
import jax
import jax.numpy as jnp
from jax.experimental import pallas as pl

B, F, NA, C, NO = 8, 6, 16384, 21, 16
THRESHOLD = 0.5
V0, V1 = 0.1, 0.2
NEG_POS_RATIO = 3
PR, PC = 128, 128


def _body(t_ref, a_ref, loc_ref, conf_ref, ll_ref, lc_ref):
    f32 = jnp.float32
    acx, acy, aw, ah = a_ref[0], a_ref[1], a_ref[2], a_ref[3]
    ax1 = acx - aw / 2.0
    ay1 = acy - ah / 2.0
    ax2 = acx + aw / 2.0
    ay2 = acy + ah / 2.0
    area_a = (ax2 - ax1) * (ay2 - ay1)
    r0 = jax.lax.broadcasted_iota(jnp.int32, (PR, PC), 0)
    c0 = jax.lax.broadcasted_iota(jnp.int32, (PR, PC), 1)
    aidx = r0 * PC + c0

    btov = jnp.full((PR, PC), -1.0, f32)
    bti = jnp.zeros((PR, PC), jnp.int32)
    bpi = []
    for o in range(NO):
        tx1 = t_ref[0, 0, o, 0]
        ty1 = t_ref[0, 0, o, 1]
        tx2 = t_ref[0, 0, o, 2]
        ty2 = t_ref[0, 0, o, 3]
        iw = jnp.clip(jnp.minimum(ax2, tx2) - jnp.maximum(ax1, tx1), 0.0, None)
        ih = jnp.clip(jnp.minimum(ay2, ty2) - jnp.maximum(ay1, ty1), 0.0, None)
        inter = iw * ih
        area_t = (tx2 - tx1) * (ty2 - ty1)
        ov = inter / (area_t + area_a - inter)
        upd = ov > btov
        btov = jnp.where(upd, ov, btov)
        bti = jnp.where(upd, o, bti)
        m = jnp.max(ov)
        bpi.append(jnp.min(jnp.where(ov == m, aidx, NA)))

    fmask = jnp.zeros((PR, PC), jnp.bool_)
    for o in range(NO):
        hit = aidx == bpi[o]
        fmask = jnp.logical_or(fmask, hit)
        bti = jnp.where(hit, o, bti)
    btov = jnp.where(fmask, 2.0, btov)

    mx1 = jnp.zeros((PR, PC), f32)
    my1 = jnp.zeros((PR, PC), f32)
    mx2 = jnp.zeros((PR, PC), f32)
    my2 = jnp.zeros((PR, PC), f32)
    lab = jnp.zeros((PR, PC), f32)
    for o in range(NO):
        sel = bti == o
        mx1 = jnp.where(sel, t_ref[0, 0, o, 0], mx1)
        my1 = jnp.where(sel, t_ref[0, 0, o, 1], my1)
        mx2 = jnp.where(sel, t_ref[0, 0, o, 2], mx2)
        my2 = jnp.where(sel, t_ref[0, 0, o, 3], my2)
        lab = jnp.where(sel, t_ref[0, 0, o, 4], lab)

    pos = btov >= THRESHOLD
    posf = pos.astype(f32)
    cls = jnp.where(pos, lab.astype(jnp.int32) + 1, 0)

    gcx = ((mx1 + mx2) / 2.0 - acx) / (V0 * aw)
    gcy = ((my1 + my2) / 2.0 - acy) / (V0 * ah)
    gw = jnp.log(jnp.clip(mx2 - mx1, 1e-6, None) / aw) / V1
    gh = jnp.log(jnp.clip(my2 - my1, 1e-6, None) / ah) / V1

    hub = jnp.float32(0.0)
    for c, g in enumerate((gcx, gcy, gw, gh)):
        d = loc_ref[0, 0, c] - g
        ad = jnp.abs(d)
        h = jnp.where(ad < 1.0, 0.5 * d * d, ad - 0.5)
        hub = hub + jnp.sum(h * posf)

    mx = conf_ref[0, 0, 0]
    for c in range(1, C):
        mx = jnp.maximum(mx, conf_ref[0, 0, c])
    s = jnp.zeros((PR, PC), f32)
    tl = jnp.zeros((PR, PC), f32)
    for c in range(C):
        x = conf_ref[0, 0, c]
        s = s + jnp.exp(x - mx)
        tl = jnp.where(cls == c, x, tl)
    ce = mx + jnp.log(s) - tl

    cepos = jnp.sum(ce * posf)
    npos = jnp.sum(pos.astype(jnp.int32))
    k = jnp.minimum(npos * NEG_POS_RATIO, NA - 1)
    mine = jnp.where(pos, 0.0, ce)

    mb = jax.lax.bitcast_convert_type(mine, jnp.int32)

    def bis(_, lohi):
        lo, hi = lohi
        mid = lo + (hi - lo) // 2
        cnt = jnp.sum((mb >= mid).astype(jnp.int32))
        ok = cnt >= k
        return (jnp.where(ok, mid, lo), jnp.where(ok, hi, mid))

    lo, _ = jax.lax.fori_loop(0, 31, bis, (jnp.int32(0), jnp.int32(0x7F800000)))
    vkth = jax.lax.bitcast_convert_type(lo, f32)
    gtm = mine > vkth
    cgt = jnp.sum(gtm.astype(f32))
    sgt = jnp.sum(jnp.where(gtm, mine, 0.0))
    topk = sgt + (k.astype(f32) - cgt) * vkth
    topk = jnp.where(k > 0, topk, 0.0)

    @pl.when((pl.program_id(0) == 0) & (pl.program_id(1) == 0))
    def _init():
        ll_ref[...] = jnp.zeros_like(ll_ref)
        lc_ref[...] = jnp.zeros_like(lc_ref)

    ll_ref[...] += hub
    lc_ref[...] += cepos + topk


def kernel(loc_data, conf_data, anchors, targets):
    loc_p = loc_data.reshape(B, F, NA, 4).transpose(0, 1, 3, 2).reshape(B, F, 4, PR, PC)
    conf_p = conf_data.reshape(B, F, NA, C).transpose(0, 1, 3, 2).reshape(B, F, C, PR, PC)
    anch_p = anchors.T.reshape(4, PR, PC)
    ll, lc = pl.pallas_call(
        _body,
        grid=(B, F),
        in_specs=[
            pl.BlockSpec((1, 1, NO, 5), lambda b, f: (b, f, 0, 0)),
            pl.BlockSpec((4, PR, PC), lambda b, f: (0, 0, 0)),
            pl.BlockSpec((1, 1, 4, PR, PC), lambda b, f: (b, f, 0, 0, 0)),
            pl.BlockSpec((1, 1, C, PR, PC), lambda b, f: (b, f, 0, 0, 0)),
        ],
        out_specs=[
            pl.BlockSpec((1, 1), lambda b, f: (0, 0)),
            pl.BlockSpec((1, 1), lambda b, f: (0, 0)),
        ],
        out_shape=[
            jax.ShapeDtypeStruct((1, 1), jnp.float32),
            jax.ShapeDtypeStruct((1, 1), jnp.float32),
        ],
    )(targets, anch_p, loc_p, conf_p)
    return (ll[0, 0], lc[0, 0])


# speedup vs baseline: 42.0103x; 1.2782x over previous
import jax
import jax.numpy as jnp
from jax.experimental import pallas as pl
from jax.experimental.pallas import tpu as pltpu

B, F, NA, C, NO = 8, 6, 16384, 21, 16
THRESHOLD = 0.5
V0, V1 = 0.1, 0.2
NEG_POS_RATIO = 3
PR, PC = 128, 128


def _body(t_ref, a_ref, loc_ref, conf_ref, ll_ref, lc_ref, mine_s, npos_s):
    f32 = jnp.float32
    acx, acy, aw, ah = a_ref[0], a_ref[1], a_ref[2], a_ref[3]
    ax1 = acx - aw / 2.0
    ay1 = acy - ah / 2.0
    ax2 = acx + aw / 2.0
    ay2 = acy + ah / 2.0
    area_a = (ax2 - ax1) * (ay2 - ay1)
    r0 = jax.lax.broadcasted_iota(jnp.int32, (PR, PC), 0)
    c0 = jax.lax.broadcasted_iota(jnp.int32, (PR, PC), 1)
    aidx = r0 * PC + c0

    btov = jnp.full((PR, PC), -1.0, f32)
    bti = jnp.zeros((PR, PC), jnp.int32)
    bpi = []
    for o in range(NO):
        tx1 = t_ref[0, 0, o, 0]
        ty1 = t_ref[0, 0, o, 1]
        tx2 = t_ref[0, 0, o, 2]
        ty2 = t_ref[0, 0, o, 3]
        iw = jnp.clip(jnp.minimum(ax2, tx2) - jnp.maximum(ax1, tx1), 0.0, None)
        ih = jnp.clip(jnp.minimum(ay2, ty2) - jnp.maximum(ay1, ty1), 0.0, None)
        inter = iw * ih
        area_t = (tx2 - tx1) * (ty2 - ty1)
        ov = inter / (area_t + area_a - inter)
        upd = ov > btov
        btov = jnp.where(upd, ov, btov)
        bti = jnp.where(upd, o, bti)
        m = jnp.max(ov)
        bpi.append(jnp.min(jnp.where(ov == m, aidx, NA)))

    fmask = jnp.zeros((PR, PC), jnp.bool_)
    for o in range(NO):
        hit = aidx == bpi[o]
        fmask = jnp.logical_or(fmask, hit)
        bti = jnp.where(hit, o, bti)
    btov = jnp.where(fmask, 2.0, btov)

    mx1 = jnp.zeros((PR, PC), f32)
    my1 = jnp.zeros((PR, PC), f32)
    mx2 = jnp.zeros((PR, PC), f32)
    my2 = jnp.zeros((PR, PC), f32)
    lab = jnp.zeros((PR, PC), f32)
    for o in range(NO):
        sel = bti == o
        mx1 = jnp.where(sel, t_ref[0, 0, o, 0], mx1)
        my1 = jnp.where(sel, t_ref[0, 0, o, 1], my1)
        mx2 = jnp.where(sel, t_ref[0, 0, o, 2], mx2)
        my2 = jnp.where(sel, t_ref[0, 0, o, 3], my2)
        lab = jnp.where(sel, t_ref[0, 0, o, 4], lab)

    pos = btov >= THRESHOLD
    posf = pos.astype(f32)
    cls = jnp.where(pos, lab.astype(jnp.int32) + 1, 0)

    gcx = ((mx1 + mx2) / 2.0 - acx) / (V0 * aw)
    gcy = ((my1 + my2) / 2.0 - acy) / (V0 * ah)
    gw = jnp.log(jnp.clip(mx2 - mx1, 1e-6, None) / aw) / V1
    gh = jnp.log(jnp.clip(my2 - my1, 1e-6, None) / ah) / V1

    hub = jnp.float32(0.0)
    for c, g in enumerate((gcx, gcy, gw, gh)):
        d = loc_ref[0, 0, c] - g
        ad = jnp.abs(d)
        h = jnp.where(ad < 1.0, 0.5 * d * d, ad - 0.5)
        hub = hub + jnp.sum(h * posf)

    mx = conf_ref[0, 0, 0]
    for c in range(1, C):
        mx = jnp.maximum(mx, conf_ref[0, 0, c])
    s = jnp.zeros((PR, PC), f32)
    tl = jnp.zeros((PR, PC), f32)
    for c in range(C):
        x = conf_ref[0, 0, c]
        s = s + jnp.exp(x - mx)
        tl = jnp.where(cls == c, x, tl)
    ce = mx + jnp.log(s) - tl

    cepos = jnp.sum(ce * posf)
    npos = jnp.sum(posf)
    mine = jnp.where(pos, 0.0, ce)

    r = pl.program_id(0) * F + pl.program_id(1)
    mine_s[pl.ds(r, 1)] = mine.reshape(1, PR, PC)
    npos_s[pl.ds(r, 1)] = jnp.full((1, PC), npos, f32)

    @pl.when((pl.program_id(0) == 0) & (pl.program_id(1) == 0))
    def _init():
        ll_ref[...] = jnp.zeros_like(ll_ref)
        lc_ref[...] = jnp.zeros_like(lc_ref)

    ll_ref[...] += hub
    lc_ref[...] += cepos

    @pl.when((pl.program_id(0) == B - 1) & (pl.program_id(1) == F - 1))
    def _mine_all():
        nrow = B * F
        kv = jnp.minimum(npos_s[:, 0:1].reshape(nrow, 1, 1) * NEG_POS_RATIO,
                         float(NA - 1)).astype(jnp.int32)
        mall = mine_s[...]
        mb = jax.lax.bitcast_convert_type(mall, jnp.int32)

        def rsum(x):
            return jnp.sum(jnp.sum(x, axis=2, keepdims=True), axis=1,
                           keepdims=True)

        def bis(_, lohi):
            lo, hi = lohi
            mid = lo + (hi - lo) // 2
            cnt = rsum((mb >= mid).astype(jnp.int32))
            ok = cnt >= kv
            return (jnp.where(ok, mid, lo), jnp.where(ok, hi, mid))

        lo0 = jnp.zeros((nrow, 1, 1), jnp.int32)
        hi0 = jnp.full((nrow, 1, 1), 0x7F800000, jnp.int32)
        lo, _ = jax.lax.fori_loop(0, 31, bis, (lo0, hi0))
        vkth = jax.lax.bitcast_convert_type(lo, f32)
        gtm = mall > vkth
        cgt = rsum(gtm.astype(f32))
        sgt = rsum(jnp.where(gtm, mall, 0.0))
        topk = sgt + (kv.astype(f32) - cgt) * vkth
        topk = jnp.where(kv > 0, topk, 0.0)
        lc_ref[...] += jnp.sum(topk)


def kernel(loc_data, conf_data, anchors, targets):
    loc_p = loc_data.reshape(B, F, NA, 4).transpose(0, 1, 3, 2).reshape(B, F, 4, PR, PC)
    conf_p = conf_data.reshape(B, F, NA, C).transpose(0, 1, 3, 2).reshape(B, F, C, PR, PC)
    anch_p = anchors.T.reshape(4, PR, PC)
    ll, lc = pl.pallas_call(
        _body,
        grid=(B, F),
        in_specs=[
            pl.BlockSpec((1, 1, NO, 5), lambda b, f: (b, f, 0, 0)),
            pl.BlockSpec((4, PR, PC), lambda b, f: (0, 0, 0)),
            pl.BlockSpec((1, 1, 4, PR, PC), lambda b, f: (b, f, 0, 0, 0)),
            pl.BlockSpec((1, 1, C, PR, PC), lambda b, f: (b, f, 0, 0, 0)),
        ],
        out_specs=[
            pl.BlockSpec((1, 1), lambda b, f: (0, 0)),
            pl.BlockSpec((1, 1), lambda b, f: (0, 0)),
        ],
        out_shape=[
            jax.ShapeDtypeStruct((1, 1), jnp.float32),
            jax.ShapeDtypeStruct((1, 1), jnp.float32),
        ],
        scratch_shapes=[
            pltpu.VMEM((B * F, PR, PC), jnp.float32),
            pltpu.VMEM((B * F, PC), jnp.float32),
        ],
    )(targets, anch_p, loc_p, conf_p)
    return (ll[0, 0], lc[0, 0])


# speedup vs baseline: 45.8380x; 1.0911x over previous
import jax
import jax.numpy as jnp
from jax.experimental import pallas as pl
from jax.experimental.pallas import tpu as pltpu

B, F, NA, C, NO = 8, 6, 16384, 21, 16
THRESHOLD = 0.5
V0, V1 = 0.1, 0.2
NEG_POS_RATIO = 3
PR, PC = 128, 128


def _body(t_ref, a_ref, loc_ref, conf_ref, ll_ref, lc_ref, mine_s, npos_s):
    f32 = jnp.float32
    acx, acy, aw, ah = a_ref[0], a_ref[1], a_ref[2], a_ref[3]
    ax1 = acx - aw / 2.0
    ay1 = acy - ah / 2.0
    ax2 = acx + aw / 2.0
    ay2 = acy + ah / 2.0
    area_a = (ax2 - ax1) * (ay2 - ay1)
    r0 = jax.lax.broadcasted_iota(jnp.int32, (PR, PC), 0)
    c0 = jax.lax.broadcasted_iota(jnp.int32, (PR, PC), 1)
    aidx = r0 * PC + c0

    btov = jnp.full((PR, PC), -1.0, f32)
    bti = jnp.zeros((PR, PC), jnp.int32)
    bpi = []
    for o in range(NO):
        tx1 = t_ref[0, 0, o, 0]
        ty1 = t_ref[0, 0, o, 1]
        tx2 = t_ref[0, 0, o, 2]
        ty2 = t_ref[0, 0, o, 3]
        iw = jnp.clip(jnp.minimum(ax2, tx2) - jnp.maximum(ax1, tx1), 0.0, None)
        ih = jnp.clip(jnp.minimum(ay2, ty2) - jnp.maximum(ay1, ty1), 0.0, None)
        inter = iw * ih
        area_t = (tx2 - tx1) * (ty2 - ty1)
        ov = inter / (area_t + area_a - inter)
        upd = ov > btov
        btov = jnp.where(upd, ov, btov)
        bti = jnp.where(upd, o, bti)
        m = jnp.max(ov)
        bpi.append(jnp.min(jnp.where(ov == m, aidx, NA)))

    fmask = jnp.zeros((PR, PC), jnp.bool_)
    for o in range(NO):
        hit = aidx == bpi[o]
        fmask = jnp.logical_or(fmask, hit)
        bti = jnp.where(hit, o, bti)
    btov = jnp.where(fmask, 2.0, btov)

    mx1 = jnp.zeros((PR, PC), f32)
    my1 = jnp.zeros((PR, PC), f32)
    mx2 = jnp.zeros((PR, PC), f32)
    my2 = jnp.zeros((PR, PC), f32)
    lab = jnp.zeros((PR, PC), f32)
    for o in range(NO):
        sel = bti == o
        mx1 = jnp.where(sel, t_ref[0, 0, o, 0], mx1)
        my1 = jnp.where(sel, t_ref[0, 0, o, 1], my1)
        mx2 = jnp.where(sel, t_ref[0, 0, o, 2], mx2)
        my2 = jnp.where(sel, t_ref[0, 0, o, 3], my2)
        lab = jnp.where(sel, t_ref[0, 0, o, 4], lab)

    pos = btov >= THRESHOLD
    posf = pos.astype(f32)
    cls = jnp.where(pos, lab.astype(jnp.int32) + 1, 0)

    gcx = ((mx1 + mx2) / 2.0 - acx) / (V0 * aw)
    gcy = ((my1 + my2) / 2.0 - acy) / (V0 * ah)
    gw = jnp.log(jnp.clip(mx2 - mx1, 1e-6, None) / aw) / V1
    gh = jnp.log(jnp.clip(my2 - my1, 1e-6, None) / ah) / V1

    hub = jnp.float32(0.0)
    for c, g in enumerate((gcx, gcy, gw, gh)):
        d = loc_ref[0, 0, c] - g
        ad = jnp.abs(d)
        h = jnp.where(ad < 1.0, 0.5 * d * d, ad - 0.5)
        hub = hub + jnp.sum(h * posf)

    mx = conf_ref[0, 0, 0]
    for c in range(1, C):
        mx = jnp.maximum(mx, conf_ref[0, 0, c])
    s = jnp.zeros((PR, PC), f32)
    tl = jnp.zeros((PR, PC), f32)
    for c in range(C):
        x = conf_ref[0, 0, c]
        s = s + jnp.exp(x - mx)
        tl = jnp.where(cls == c, x, tl)
    ce = mx + jnp.log(s) - tl

    cepos = jnp.sum(ce * posf)
    npos = jnp.sum(posf)
    mine = jnp.where(pos, 0.0, ce)

    r = pl.program_id(0) * F + pl.program_id(1)
    mine_s[pl.ds(r, 1)] = mine.reshape(1, PR, PC)
    npos_s[pl.ds(r, 1)] = jnp.full((1, PC), npos, f32)

    @pl.when((pl.program_id(0) == 0) & (pl.program_id(1) == 0))
    def _init():
        ll_ref[...] = jnp.zeros_like(ll_ref)
        lc_ref[...] = jnp.zeros_like(lc_ref)

    ll_ref[...] += hub
    lc_ref[...] += cepos

    @pl.when((pl.program_id(0) == B - 1) & (pl.program_id(1) == F - 1))
    def _mine_all():
        nrow = B * F
        kv = jnp.minimum(npos_s[:, 0:1].reshape(nrow, 1, 1) * NEG_POS_RATIO,
                         float(NA - 1))
        mall = mine_s[...]
        mb = jax.lax.bitcast_convert_type(mall, jnp.int32)

        def rsum(x):
            return jnp.sum(jnp.sum(x, axis=1, keepdims=True), axis=2,
                           keepdims=True)

        def bis(_, lohi):
            lo, hi = lohi
            mid = lo + (hi - lo) // 2
            cnt = rsum(jnp.where(mb >= mid, 1.0, 0.0))
            ok = cnt >= kv
            return (jnp.where(ok, mid, lo), jnp.where(ok, hi, mid))

        lo0 = jnp.zeros((nrow, 1, 1), jnp.int32)
        hi0 = jnp.full((nrow, 1, 1), 0x7F800000, jnp.int32)
        lo, _ = jax.lax.fori_loop(0, 31, bis, (lo0, hi0))
        vkth = jax.lax.bitcast_convert_type(lo, f32)
        gtm = mall > vkth
        cgt = rsum(jnp.where(gtm, 1.0, 0.0))
        sgt = rsum(jnp.where(gtm, mall, 0.0))
        topk = sgt + (kv - cgt) * vkth
        topk = jnp.where(kv > 0, topk, 0.0)
        lc_ref[...] += jnp.sum(topk)


def kernel(loc_data, conf_data, anchors, targets):
    loc_p = loc_data.reshape(B, F, NA, 4).transpose(0, 1, 3, 2).reshape(B, F, 4, PR, PC)
    conf_p = conf_data.reshape(B, F, NA, C).transpose(0, 1, 3, 2).reshape(B, F, C, PR, PC)
    anch_p = anchors.T.reshape(4, PR, PC)
    ll, lc = pl.pallas_call(
        _body,
        grid=(B, F),
        in_specs=[
            pl.BlockSpec((1, 1, NO, 5), lambda b, f: (b, f, 0, 0)),
            pl.BlockSpec((4, PR, PC), lambda b, f: (0, 0, 0)),
            pl.BlockSpec((1, 1, 4, PR, PC), lambda b, f: (b, f, 0, 0, 0)),
            pl.BlockSpec((1, 1, C, PR, PC), lambda b, f: (b, f, 0, 0, 0)),
        ],
        out_specs=[
            pl.BlockSpec((1, 1), lambda b, f: (0, 0)),
            pl.BlockSpec((1, 1), lambda b, f: (0, 0)),
        ],
        out_shape=[
            jax.ShapeDtypeStruct((1, 1), jnp.float32),
            jax.ShapeDtypeStruct((1, 1), jnp.float32),
        ],
        scratch_shapes=[
            pltpu.VMEM((B * F, PR, PC), jnp.float32),
            pltpu.VMEM((B * F, PC), jnp.float32),
        ],
    )(targets, anch_p, loc_p, conf_p)
    return (ll[0, 0], lc[0, 0])
